# Initial kernel scaffold; baseline (speedup 1.0000x reference)
import jax
import jax.numpy as jnp
from jax.experimental import pallas as pl


def kernel(user, item, theta_table, a_table, b_table, c_table, W1, b1, W2, b2, W3, b3, Wd, bd):
    raise NotImplementedError("write your pallas kernel here")


# speedup vs baseline: 1.0348x; 1.0348x over previous
import functools

import jax
import jax.numpy as jnp
from jax import lax
from jax.experimental import pallas as pl
from jax.experimental.pallas import tpu as pltpu
from jax.experimental.pallas import tpu_sc as plsc

_B = 16384
_NC = 2
_NS = 16
_NW = _NC * _NS
_LANES = 16
_BPW = _B // _NW
_D = 1.702
_VALUE_RANGE = 8.0
_A_RANGE = 3.0


def _sigmoid(x):
    return 1.0 / (1.0 + jnp.exp(-x))


def _sc_body(user_hbm, item_hbm, theta_hbm, a_hbm, b_hbm, c_hbm,
             irf_hbm, theta_raw_hbm,
             uidx_v, iidx_v, th_v, a_v, b_v, c_v, irf_v, sem):
    wid = lax.axis_index("s") * _NC + lax.axis_index("c")
    base = wid * _BPW
    pltpu.sync_copy(user_hbm.at[pl.ds(base, _BPW)], uidx_v)
    pltpu.sync_copy(item_hbm.at[pl.ds(base, _BPW)], iidx_v)
    cps = [
        pltpu.async_copy(theta_hbm.at[uidx_v], th_v, sem),
        pltpu.async_copy(a_hbm.at[iidx_v], a_v, sem),
        pltpu.async_copy(b_hbm.at[iidx_v], b_v, sem),
        pltpu.async_copy(c_hbm.at[iidx_v], c_v, sem),
    ]
    for cp in cps:
        cp.wait()

    def step(i, _):
        sl = pl.ds(i * _LANES, _LANES)
        th = th_v[sl]
        a = a_v[sl]
        b = b_v[sl]
        c = c_v[sl]
        c_s = _sigmoid(c)
        theta_t = _VALUE_RANGE * (_sigmoid(th) - 0.5)
        b_t = _VALUE_RANGE * (_sigmoid(b) - 0.5)
        a_t = _A_RANGE * _sigmoid(a)
        irf = c_s + (1.0 - c_s) / (1.0 + jnp.exp(-_D * a_t * (theta_t - b_t)))
        irf_v[sl] = irf
        return 0

    lax.fori_loop(0, _BPW // _LANES, step, 0)
    pltpu.sync_copy(irf_v, irf_hbm.at[pl.ds(base, _BPW)])
    pltpu.sync_copy(th_v, theta_raw_hbm.at[pl.ds(base, _BPW)])


_sc_gather_irf = pl.kernel(
    _sc_body,
    out_type=[
        jax.ShapeDtypeStruct((_B,), jnp.float32),
        jax.ShapeDtypeStruct((_B,), jnp.float32),
    ],
    mesh=plsc.VectorSubcoreMesh(core_axis_name="c", subcore_axis_name="s"),
    scratch_types=[
        pltpu.VMEM((_BPW,), jnp.int32),
        pltpu.VMEM((_BPW,), jnp.int32),
        pltpu.VMEM((_BPW,), jnp.float32),
        pltpu.VMEM((_BPW,), jnp.float32),
        pltpu.VMEM((_BPW,), jnp.float32),
        pltpu.VMEM((_BPW,), jnp.float32),
        pltpu.VMEM((_BPW,), jnp.float32),
        pltpu.SemaphoreType.DMA,
    ],
)


def _mlp_body(x_ref, w1_ref, b1_ref, w2_ref, b2_ref, w3_ref, b3_ref,
              wd_ref, bd_ref, o_ref):
    x = x_ref[...]
    h1 = jnp.maximum(x * w1_ref[...] + b1_ref[...], 0.0)
    h2 = jnp.maximum(
        jnp.dot(h1, w2_ref[...], preferred_element_type=jnp.float32)
        + b2_ref[...], 0.0)
    h3 = jnp.dot(h2, w3_ref[...], preferred_element_type=jnp.float32)
    o_ref[...] = (h3 + b3_ref[0, 0]) * wd_ref[0, 0] + bd_ref[0, 0]


_mlp = pl.pallas_call(
    _mlp_body,
    out_shape=jax.ShapeDtypeStruct((_B, 1), jnp.float32),
)


def kernel(user, item, theta_table, a_table, b_table, c_table,
           W1, b1, W2, b2, W3, b3, Wd, bd):
    irf_out, theta_raw = _sc_gather_irf(
        user, item,
        theta_table.reshape(-1), a_table.reshape(-1),
        b_table.reshape(-1), c_table.reshape(-1))
    d_output = _mlp(
        theta_raw.reshape(_B, 1),
        W1.reshape(1, 64), b1.reshape(1, 64),
        W2.T, b2.reshape(1, 32),
        W3.T, b3.reshape(1, 1),
        Wd.reshape(1, 1), bd.reshape(1, 1))
    return (irf_out, d_output)


# speedup vs baseline: 1.3323x; 1.2874x over previous
import jax
import jax.numpy as jnp
from jax import lax
from jax.experimental import pallas as pl
from jax.experimental.pallas import tpu as pltpu
from jax.experimental.pallas import tpu_sc as plsc

_B = 16384
_NC = 2
_NS = 16
_NW = _NC * _NS
_LANES = 16
_BPW = _B // _NW
_D = 1.702
_VALUE_RANGE = 8.0
_A_RANGE = 3.0


def _sigmoid(x):
    return 1.0 / (1.0 + jnp.exp(-x))


def _sc_body(user_hbm, item_hbm, theta_hbm, a_hbm, b_hbm, c_hbm,
             irf_hbm, theta_raw_hbm,
             uidx_v, iidx_v, th_v, a_v, b_v, c_v, irf_v, sem):
    wid = lax.axis_index("s") * _NC + lax.axis_index("c")
    base = wid * _BPW
    pltpu.sync_copy(user_hbm.at[pl.ds(base, _BPW)], uidx_v)
    pltpu.sync_copy(item_hbm.at[pl.ds(base, _BPW)], iidx_v)
    cps = [
        pltpu.async_copy(theta_hbm.at[uidx_v], th_v, sem),
        pltpu.async_copy(a_hbm.at[iidx_v], a_v, sem),
        pltpu.async_copy(b_hbm.at[iidx_v], b_v, sem),
        pltpu.async_copy(c_hbm.at[iidx_v], c_v, sem),
    ]
    for cp in cps:
        cp.wait()

    def step(i, _):
        sl = pl.ds(i * _LANES, _LANES)
        th = th_v[sl]
        a = a_v[sl]
        b = b_v[sl]
        c = c_v[sl]
        c_s = _sigmoid(c)
        theta_t = _VALUE_RANGE * (_sigmoid(th) - 0.5)
        b_t = _VALUE_RANGE * (_sigmoid(b) - 0.5)
        a_t = _A_RANGE * _sigmoid(a)
        irf = c_s + (1.0 - c_s) / (1.0 + jnp.exp(-_D * a_t * (theta_t - b_t)))
        irf_v[sl] = irf
        return 0

    lax.fori_loop(0, _BPW // _LANES, step, 0)
    pltpu.sync_copy(irf_v, irf_hbm.at[pl.ds(base, _BPW)])
    pltpu.sync_copy(th_v, theta_raw_hbm.at[pl.ds(base, _BPW)])


_sc_gather_irf = pl.kernel(
    _sc_body,
    out_type=[
        jax.ShapeDtypeStruct((_B,), jnp.float32),
        jax.ShapeDtypeStruct((_B,), jnp.float32),
    ],
    mesh=plsc.VectorSubcoreMesh(core_axis_name="c", subcore_axis_name="s"),
    scratch_types=[
        pltpu.VMEM((_BPW,), jnp.int32),
        pltpu.VMEM((_BPW,), jnp.int32),
        pltpu.VMEM((_BPW,), jnp.float32),
        pltpu.VMEM((_BPW,), jnp.float32),
        pltpu.VMEM((_BPW,), jnp.float32),
        pltpu.VMEM((_BPW,), jnp.float32),
        pltpu.VMEM((_BPW,), jnp.float32),
        pltpu.SemaphoreType.DMA,
    ],
)


def _mlp_body(x_ref, w1_ref, b1_ref, w2_ref, b2_ref, w3_ref, b3_ref,
              wd_ref, bd_ref, d_ref):
    x = x_ref[...]
    h1 = jnp.maximum(w1_ref[...] * x + b1_ref[...], 0.0)
    h2 = jnp.maximum(
        jnp.dot(w2_ref[...], h1, preferred_element_type=jnp.float32)
        + b2_ref[...], 0.0)
    h3 = jnp.sum(w3_ref[...] * h2, axis=0, keepdims=True)
    d_ref[...] = (h3 + b3_ref[0, 0]) * wd_ref[0, 0] + bd_ref[0, 0]


_mlp = pl.pallas_call(
    _mlp_body,
    out_shape=jax.ShapeDtypeStruct((1, _B), jnp.float32),
)


def kernel(user, item, theta_table, a_table, b_table, c_table,
           W1, b1, W2, b2, W3, b3, Wd, bd):
    irf_out, theta_raw = _sc_gather_irf(
        user, item,
        theta_table.reshape(-1), a_table.reshape(-1),
        b_table.reshape(-1), c_table.reshape(-1))
    d_r = _mlp(
        theta_raw.reshape(1, _B),
        W1, b1.reshape(64, 1),
        W2, b2.reshape(32, 1),
        W3.T, b3.reshape(1, 1),
        Wd, bd.reshape(1, 1))
    return (irf_out, d_r.reshape(_B, 1))
